# Initial kernel scaffold; baseline (speedup 1.0000x reference)
import jax
import jax.numpy as jnp
from jax.experimental import pallas as pl


def kernel(x, sample_map, output_height, output_width):
    raise NotImplementedError("write your pallas kernel here")


# speedup vs baseline: 16.5411x; 16.5411x over previous
import jax
import jax.numpy as jnp
from jax import lax
from jax.experimental import pallas as pl
from jax.experimental.pallas import tpu as pltpu
from jax.experimental.pallas import tpu_sc as plsc

OH_, OW_ = 384, 384
B_, C_ = 2, 96
HW = 384 * 384
NP = B_ * C_
NC, NS, L = 2, 16, 16
PX = HW // NS
NG = PX // L
NI = 4 * PX // (32 * 128)


def _sc_body(x_hbm, mx_hbm, my_hbm, out_hbm,
             idx3, contrib3, x_v, wx_v, wy_v, z_v, table):
    c = lax.axis_index("c")
    s = lax.axis_index("s")
    px_base = s * PX

    pltpu.sync_copy(mx_hbm.at[pl.ds(px_base, PX)], wx_v)
    pltpu.sync_copy(my_hbm.at[pl.ds(px_base, PX)], wy_v)

    def init_group(i, carry):
        d0 = i // 64
        d1 = (i % 64) // 2
        col = (i % 2) * 64
        off = i * L
        mx = wx_v[pl.ds(off, L)]
        my = wy_v[pl.ds(off, L)]
        x0i = mx.astype(jnp.int32)
        y0i = my.astype(jnp.int32)
        wx1 = mx - x0i.astype(jnp.float32)
        wy1 = my - y0i.astype(jnp.float32)
        zero = jnp.zeros((L,), jnp.int32)
        maxw = jnp.full((L,), OW_ - 1, jnp.int32)
        maxh = jnp.full((L,), OH_ - 1, jnp.int32)
        x0 = jnp.minimum(jnp.maximum(x0i, zero), maxw)
        x1 = jnp.minimum(jnp.maximum(x0i + 1, zero), maxw)
        y0 = jnp.minimum(jnp.maximum(y0i, zero), maxh)
        y1 = jnp.minimum(jnp.maximum(y0i + 1, zero), maxh)
        idx3[d0, d1, pl.ds(col, L)] = y0 * OW_ + x0
        idx3[d0, d1, pl.ds(col + 16, L)] = y0 * OW_ + x1
        idx3[d0, d1, pl.ds(col + 32, L)] = y1 * OW_ + x0
        idx3[d0, d1, pl.ds(col + 48, L)] = y1 * OW_ + x1
        wx_v[pl.ds(off, L)] = wx1
        wy_v[pl.ds(off, L)] = wy1
        z_v[pl.ds(off, L)] = jnp.zeros((L,), jnp.float32)
        return carry

    lax.fori_loop(0, NG, init_group, 0)

    pltpu.sync_copy(z_v, table.at[pl.ds(px_base, PX)])

    def plane_body(pp, carry):
        plane = c * (NP // NC) + pp
        pltpu.sync_copy(x_hbm.at[plane, pl.ds(px_base, PX)], x_v)

        def comp_group(i, cc):
            d0 = i // 64
            d1 = (i % 64) // 2
            col = (i % 2) * 64
            off = i * L
            v = x_v[pl.ds(off, L)]
            wx1 = wx_v[pl.ds(off, L)]
            wy1 = wy_v[pl.ds(off, L)]
            one = jnp.ones((L,), jnp.float32)
            vy0 = v * (one - wy1)
            vy1 = v * wy1
            contrib3[d0, d1, pl.ds(col, L)] = vy0 * (one - wx1)
            contrib3[d0, d1, pl.ds(col + 16, L)] = vy0 * wx1
            contrib3[d0, d1, pl.ds(col + 32, L)] = vy1 * (one - wx1)
            contrib3[d0, d1, pl.ds(col + 48, L)] = vy1 * wx1
            return cc

        lax.fori_loop(0, NG, comp_group, 0)

        plsc.subcore_barrier()

        def issue(j, cc):
            d0 = j // 32
            d1 = j % 32
            pltpu.sync_copy(contrib3.at[d0, d1],
                            table.at[idx3.at[d0, d1]], add=True)
            return cc

        lax.fori_loop(0, NI * 32, issue, 0)

        plsc.subcore_barrier()

        pltpu.sync_copy(table.at[pl.ds(px_base, PX)],
                        out_hbm.at[plane, pl.ds(px_base, PX)])
        pltpu.sync_copy(z_v, table.at[pl.ds(px_base, PX)])
        return carry

    lax.fori_loop(0, NP // NC, plane_body, 0)


@jax.jit
def _splat(x2, mx, my):
    mesh = plsc.VectorSubcoreMesh(core_axis_name="c", subcore_axis_name="s")
    return pl.kernel(
        _sc_body,
        out_type=jax.ShapeDtypeStruct((NP, HW), jnp.float32),
        mesh=mesh,
        scratch_types=[
            pltpu.VMEM((NI, 32, 128), jnp.int32),
            pltpu.VMEM((NI, 32, 128), jnp.float32),
            pltpu.VMEM((PX,), jnp.float32),
            pltpu.VMEM((PX,), jnp.float32),
            pltpu.VMEM((PX,), jnp.float32),
            pltpu.VMEM((PX,), jnp.float32),
            pltpu.VMEM_SHARED((HW,), jnp.float32),
        ],
    )(x2, mx, my)


def kernel(x, sample_map, output_height, output_width):
    Bn, Cn, Hn, Wn = x.shape
    x2 = x.reshape(Bn * Cn, Hn * Wn)
    mx = sample_map[..., 0].reshape(-1)
    my = sample_map[..., 1].reshape(-1)
    out2 = _splat(x2, mx, my)
    return out2.reshape(Bn, Cn, OH_, OW_)


# speedup vs baseline: 30.3848x; 1.8369x over previous
import jax
import jax.numpy as jnp
from jax import lax
from jax.experimental import pallas as pl
from jax.experimental.pallas import tpu as pltpu
from jax.experimental.pallas import tpu_sc as plsc

OH_, OW_ = 384, 384
B_, C_ = 2, 96
HW = 384 * 384
NP = B_ * C_
NC, NS, L = 2, 16, 16
PX = HW // NS
NG = PX // L
NI = 4 * PX // (32 * 128)


def _sc_body(x_hbm, mx_hbm, my_hbm, out_hbm,
             idx1, contrib1, x_v, wx_v, wy_v, z_v, table):
    c = lax.axis_index("c")
    s = lax.axis_index("s")
    px_base = s * PX

    pltpu.sync_copy(mx_hbm.at[pl.ds(px_base, PX)], wx_v)
    pltpu.sync_copy(my_hbm.at[pl.ds(px_base, PX)], wy_v)

    def init_group(i, carry):
        f = i * 64
        off = i * L
        mx = wx_v[pl.ds(off, L)]
        my = wy_v[pl.ds(off, L)]
        x0i = mx.astype(jnp.int32)
        y0i = my.astype(jnp.int32)
        wx1 = mx - x0i.astype(jnp.float32)
        wy1 = my - y0i.astype(jnp.float32)
        zero = jnp.zeros((L,), jnp.int32)
        maxw = jnp.full((L,), OW_ - 1, jnp.int32)
        maxh = jnp.full((L,), OH_ - 1, jnp.int32)
        x0 = jnp.minimum(jnp.maximum(x0i, zero), maxw)
        x1 = jnp.minimum(jnp.maximum(x0i + 1, zero), maxw)
        y0 = jnp.minimum(jnp.maximum(y0i, zero), maxh)
        y1 = jnp.minimum(jnp.maximum(y0i + 1, zero), maxh)
        idx1[pl.ds(f, L)] = y0 * OW_ + x0
        idx1[pl.ds(f + 16, L)] = y0 * OW_ + x1
        idx1[pl.ds(f + 32, L)] = y1 * OW_ + x0
        idx1[pl.ds(f + 48, L)] = y1 * OW_ + x1
        wx_v[pl.ds(off, L)] = wx1
        wy_v[pl.ds(off, L)] = wy1
        z_v[pl.ds(off, L)] = jnp.zeros((L,), jnp.float32)
        return carry

    lax.fori_loop(0, NG, init_group, 0)

    pltpu.sync_copy(z_v, table.at[pl.ds(px_base, PX)])

    def plane_body(pp, carry):
        plane = c * (NP // NC) + pp
        pltpu.sync_copy(x_hbm.at[plane, pl.ds(px_base, PX)], x_v)

        def comp_group(i, cc):
            f = i * 64
            off = i * L
            v = x_v[pl.ds(off, L)]
            wx1 = wx_v[pl.ds(off, L)]
            wy1 = wy_v[pl.ds(off, L)]
            one = jnp.ones((L,), jnp.float32)
            vy0 = v * (one - wy1)
            vy1 = v * wy1
            contrib1[pl.ds(f, L)] = vy0 * (one - wx1)
            contrib1[pl.ds(f + 16, L)] = vy0 * wx1
            contrib1[pl.ds(f + 32, L)] = vy1 * (one - wx1)
            contrib1[pl.ds(f + 48, L)] = vy1 * wx1
            return cc

        lax.fori_loop(0, NG, comp_group, 0)

        plsc.subcore_barrier()

        pltpu.sync_copy(contrib1, table.at[idx1], add=True)

        plsc.subcore_barrier()

        pltpu.sync_copy(table.at[pl.ds(px_base, PX)],
                        out_hbm.at[plane, pl.ds(px_base, PX)])
        pltpu.sync_copy(z_v, table.at[pl.ds(px_base, PX)])
        return carry

    lax.fori_loop(0, NP // NC, plane_body, 0)


@jax.jit
def _splat(x2, mx, my):
    mesh = plsc.VectorSubcoreMesh(core_axis_name="c", subcore_axis_name="s")
    return pl.kernel(
        _sc_body,
        out_type=jax.ShapeDtypeStruct((NP, HW), jnp.float32),
        mesh=mesh,
        scratch_types=[
            pltpu.VMEM((4 * PX,), jnp.int32),
            pltpu.VMEM((4 * PX,), jnp.float32),
            pltpu.VMEM((PX,), jnp.float32),
            pltpu.VMEM((PX,), jnp.float32),
            pltpu.VMEM((PX,), jnp.float32),
            pltpu.VMEM((PX,), jnp.float32),
            pltpu.VMEM_SHARED((HW,), jnp.float32),
        ],
    )(x2, mx, my)


def kernel(x, sample_map, output_height, output_width):
    Bn, Cn, Hn, Wn = x.shape
    x2 = x.reshape(Bn * Cn, Hn * Wn)
    mx = sample_map[..., 0].reshape(-1)
    my = sample_map[..., 1].reshape(-1)
    out2 = _splat(x2, mx, my)
    return out2.reshape(Bn, Cn, OH_, OW_)


# speedup vs baseline: 42.9809x; 1.4146x over previous
import jax
import jax.numpy as jnp
from jax import lax
from jax.experimental import pallas as pl
from jax.experimental.pallas import tpu as pltpu
from jax.experimental.pallas import tpu_sc as plsc

OH_, OW_ = 384, 384
B_, C_ = 2, 96
HW = 384 * 384
NP = B_ * C_
NC, NS, L = 2, 16, 16
PX = HW // NS
NQ = 4
QPX = PX // NQ
QG = QPX // L
NPC = NP // NC


def _sc_body(x_hbm, mx_hbm, my_hbm, out_hbm,
             idx_q0, idx_q1, idx_q2, idx_q3, c_q0, c_q1, c_q2, c_q3,
             x_v0, x_v1, wx_v, wy_v, z_v, table,
             s_x0, s_x1, s_sc, s_dr):
    c = lax.axis_index("c")
    s = lax.axis_index("s")
    px_base = s * PX
    base = c * NPC
    idx_qs = [idx_q0, idx_q1, idx_q2, idx_q3]
    c_qs = [c_q0, c_q1, c_q2, c_q3]

    pltpu.sync_copy(mx_hbm.at[pl.ds(px_base, PX)], x_v0)
    pltpu.sync_copy(my_hbm.at[pl.ds(px_base, PX)], x_v1)

    def init_group(q):
        idx_b = idx_qs[q]
        def body(i, carry):
            g = q * QG + i
            off0 = g * L
            mx = x_v0[pl.ds(off0, L)]
            my = x_v1[pl.ds(off0, L)]
            x0i = mx.astype(jnp.int32)
            y0i = my.astype(jnp.int32)
            wx1 = mx - x0i.astype(jnp.float32)
            wy1 = my - y0i.astype(jnp.float32)
            zero = jnp.zeros((L,), jnp.int32)
            maxw = jnp.full((L,), OW_ - 1, jnp.int32)
            maxh = jnp.full((L,), OH_ - 1, jnp.int32)
            x0 = jnp.minimum(jnp.maximum(x0i, zero), maxw)
            x1 = jnp.minimum(jnp.maximum(x0i + 1, zero), maxw)
            y0 = jnp.minimum(jnp.maximum(y0i, zero), maxh)
            y1 = jnp.minimum(jnp.maximum(y0i + 1, zero), maxh)
            fl = i * (4 * L)
            idx_b[pl.ds(fl, L)] = y0 * OW_ + x0
            idx_b[pl.ds(fl + L, L)] = y0 * OW_ + x1
            idx_b[pl.ds(fl + 2 * L, L)] = y1 * OW_ + x0
            idx_b[pl.ds(fl + 3 * L, L)] = y1 * OW_ + x1
            off = g * L
            wx_v[pl.ds(off, L)] = wx1
            wy_v[pl.ds(off, L)] = wy1
            z_v[pl.ds(off, L)] = jnp.zeros((L,), jnp.float32)
            return carry
        return body

    for q in range(NQ):
        lax.fori_loop(0, QG, init_group(q), 0)

    pltpu.sync_copy(z_v, table.at[pl.ds(px_base, PX)])
    pltpu.async_copy(table.at[pl.ds(px_base, PX)],
                     out_hbm.at[base, pl.ds(px_base, PX)], s_dr)

    pltpu.async_copy(x_hbm.at[base, pl.ds(px_base, PX)], x_v0, s_x0)
    pltpu.async_copy(x_hbm.at[base + 1, pl.ds(px_base, PX)], x_v1, s_x1)

    def do_plane(plane, x_v, s_x):
        pltpu.make_async_copy(
            x_hbm.at[plane, pl.ds(px_base, PX)], x_v, s_x).wait()

        def comp_quarter(q):
            c_b = c_qs[q]
            def body(i, carry):
                off = (q * QG + i) * L
                fl = i * (4 * L)
                v = x_v[pl.ds(off, L)]
                wx1 = wx_v[pl.ds(off, L)]
                wy1 = wy_v[pl.ds(off, L)]
                one = jnp.ones((L,), jnp.float32)
                vy0 = v * (one - wy1)
                vy1 = v * wy1
                c_b[pl.ds(fl, L)] = vy0 * (one - wx1)
                c_b[pl.ds(fl + L, L)] = vy0 * wx1
                c_b[pl.ds(fl + 2 * L, L)] = vy1 * (one - wx1)
                c_b[pl.ds(fl + 3 * L, L)] = vy1 * wx1
                return carry
            lax.fori_loop(0, QG, body, 0)

        comp_quarter(0)
        pltpu.make_async_copy(
            table.at[pl.ds(px_base, PX)],
            out_hbm.at[0, pl.ds(px_base, PX)],
            s_dr).wait()
        pltpu.sync_copy(z_v, table.at[pl.ds(px_base, PX)])
        plsc.subcore_barrier()

        descs = [pltpu.async_copy(c_qs[0], table.at[idx_qs[0]],
                                  s_sc, add=True)]
        for q in range(1, NQ):
            comp_quarter(q)
            descs.append(pltpu.async_copy(c_qs[q], table.at[idx_qs[q]],
                                          s_sc, add=True))
        pltpu.async_copy(
            x_hbm.at[(plane + 2) % NP, pl.ds(px_base, PX)],
            x_v, s_x)
        for d in descs:
            d.wait()
        plsc.subcore_barrier()
        pltpu.async_copy(table.at[pl.ds(px_base, PX)],
                         out_hbm.at[plane, pl.ds(px_base, PX)], s_dr)

    def plane_pair(i, carry):
        do_plane(base + 2 * i, x_v0, s_x0)
        do_plane(base + 2 * i + 1, x_v1, s_x1)
        return carry

    lax.fori_loop(0, NPC // 2, plane_pair, 0)

    pltpu.make_async_copy(
        table.at[pl.ds(px_base, PX)],
        out_hbm.at[base + NPC - 1, pl.ds(px_base, PX)], s_dr).wait()
    pltpu.make_async_copy(
        x_hbm.at[0, pl.ds(px_base, PX)], x_v0, s_x0).wait()
    pltpu.make_async_copy(
        x_hbm.at[0, pl.ds(px_base, PX)], x_v1, s_x1).wait()


@jax.jit
def _splat(x2, mx, my):
    mesh = plsc.VectorSubcoreMesh(core_axis_name="c", subcore_axis_name="s")
    return pl.kernel(
        _sc_body,
        out_type=jax.ShapeDtypeStruct((NP, HW), jnp.float32),
        mesh=mesh,
        scratch_types=[
            pltpu.VMEM((4 * QPX,), jnp.int32),
            pltpu.VMEM((4 * QPX,), jnp.int32),
            pltpu.VMEM((4 * QPX,), jnp.int32),
            pltpu.VMEM((4 * QPX,), jnp.int32),
            pltpu.VMEM((4 * QPX,), jnp.float32),
            pltpu.VMEM((4 * QPX,), jnp.float32),
            pltpu.VMEM((4 * QPX,), jnp.float32),
            pltpu.VMEM((4 * QPX,), jnp.float32),
            pltpu.VMEM((PX,), jnp.float32),
            pltpu.VMEM((PX,), jnp.float32),
            pltpu.VMEM((PX,), jnp.float32),
            pltpu.VMEM((PX,), jnp.float32),
            pltpu.VMEM((PX,), jnp.float32),
            pltpu.VMEM_SHARED((HW,), jnp.float32),
            pltpu.SemaphoreType.DMA,
            pltpu.SemaphoreType.DMA,
            pltpu.SemaphoreType.DMA,
            pltpu.SemaphoreType.DMA,
        ],
    )(x2, mx, my)


def kernel(x, sample_map, output_height, output_width):
    Bn, Cn, Hn, Wn = x.shape
    x2 = x.reshape(Bn * Cn, Hn * Wn)
    mx = sample_map[..., 0].reshape(-1)
    my = sample_map[..., 1].reshape(-1)
    out2 = _splat(x2, mx, my)
    return out2.reshape(Bn, Cn, OH_, OW_)
